# baseline (speedup 1.0000x reference)
import functools

import jax
import jax.numpy as jnp
from jax import lax
from jax.experimental import pallas as pl
from jax.experimental.pallas import tpu as pltpu
from jax.experimental.pallas import tpu_sc as plsc

N = 50000
E = 800000
F_IN = 40
D = 64
Q = 16
NL = 5

NC = 2
NS = 16
CHUNK = 128
BW = 1568
DUMP = 32 * BW + 31
NPAD = 32 * BW + 32
ROWS_PER_TILE = NPAD // NS
E_CAP = 811008

BN_ROWS = 1000
NB = N // BN_ROWS
E8 = E_CAP // 8
EB8 = 3168
NEB8 = E8 // EB8


def _sc_edge_body(htab, e4, src, dst, meta, out, idx_s, idx_d, ebuf, gbuf,
                  bounce, mrow, agg_s, sem):
    c = lax.axis_index("c")
    s = lax.axis_index("s")
    zvec = jnp.zeros((16,), jnp.float32)
    r0 = s * ROWS_PER_TILE

    pltpu.sync_copy(meta.at[pl.ds(s * 16, 16)], mrow)
    nck0 = mrow[:][0]
    pltpu.sync_copy(meta.at[pl.ds(512 + s * 16, 16)], mrow)
    off0 = pl.multiple_of(mrow[:][0], CHUNK)
    pltpu.sync_copy(meta.at[pl.ds((s + 16) * 16, 16)], mrow)
    nck1 = mrow[:][0]
    pltpu.sync_copy(meta.at[pl.ds(512 + (s + 16) * 16, 16)], mrow)
    off1 = pl.multiple_of(mrow[:][0], CHUNK)

    @pl.loop(0, ROWS_PER_TILE)
    def _z(i):
        bounce[i, :] = zvec

    @pl.loop(0, 2)
    def _pass(p):
        q = c * 2 + p
        qn = q * N

        pltpu.sync_copy(bounce, agg_s.at[pl.ds(r0, ROWS_PER_TILE)])
        plsc.subcore_barrier()

        @pl.loop(0, nck0 + nck1)
        def _chunk(i):
            second = i >= nck0
            base = jnp.where(second, off1 + (i - nck0) * CHUNK,
                             off0 + i * CHUNK)
            base = pl.multiple_of(base, CHUNK)
            pltpu.sync_copy(src.at[pl.ds(base, CHUNK)], idx_s)
            pltpu.sync_copy(dst.at[pl.ds(base, CHUNK)], idx_d)
            pltpu.sync_copy(e4.at[pl.ds(q * E_CAP + base, CHUNK)], ebuf)

            @pl.loop(0, CHUNK // 16)
            def _adj(k):
                idx_s[pl.ds(k * 16, 16)] = idx_s[pl.ds(k * 16, 16)] + qn

            pltpu.async_copy(htab.at[idx_s], gbuf, sem).wait()

            @pl.loop(0, CHUNK)
            def _msg(j):
                gbuf[j, :] = jnp.maximum(gbuf[j, :] + ebuf[j, :], 0.0)

            pltpu.sync_copy(gbuf, agg_s.at[idx_d], add=True)

        plsc.subcore_barrier()
        pltpu.sync_copy(agg_s.at[pl.ds(r0, ROWS_PER_TILE)],
                        out.at[pl.ds(q * NPAD + r0, ROWS_PER_TILE)])
        plsc.subcore_barrier()


_sc_edge = functools.partial(
    pl.kernel,
    out_type=jax.ShapeDtypeStruct((4 * NPAD, Q), jnp.float32),
    mesh=plsc.VectorSubcoreMesh(core_axis_name="c", subcore_axis_name="s",
                                num_cores=NC, num_subcores=NS),
    scratch_types=[
        pltpu.VMEM((CHUNK,), jnp.int32),
        pltpu.VMEM((CHUNK,), jnp.int32),
        pltpu.VMEM((CHUNK, Q), jnp.float32),
        pltpu.VMEM((CHUNK, Q), jnp.float32),
        pltpu.VMEM((ROWS_PER_TILE, Q), jnp.float32),
        pltpu.VMEM((16,), jnp.int32),
        pltpu.VMEM_SHARED((NPAD, Q), jnp.float32),
        pltpu.SemaphoreType.DMA,
    ],
    compiler_params=pltpu.CompilerParams(use_tc_tiling_on_sc=False),
)(_sc_edge_body)


def _dot_bf16(a, b):
    return jnp.dot(a.astype(jnp.bfloat16), b.astype(jnp.bfloat16),
                   preferred_element_type=jnp.float32)


def _split4(y, out_ref):
    out_ref[0] = y[:, 0 * Q:1 * Q]
    out_ref[1] = y[:, 1 * Q:2 * Q]
    out_ref[2] = y[:, 2 * Q:3 * Q]
    out_ref[3] = y[:, 3 * Q:4 * Q]


def _embed_body(x_ref, w_ref, b_ref, out_ref):
    y = _dot_bf16(x_ref[...], w_ref[...])
    _split4(jnp.maximum(y + b_ref[...], 0.0), out_ref)


_embed = pl.pallas_call(
    _embed_body,
    grid=(NB,),
    in_specs=[
        pl.BlockSpec((BN_ROWS, F_IN), lambda j: (j, 0)),
        pl.BlockSpec((F_IN, D), lambda j: (0, 0)),
        pl.BlockSpec((1, D), lambda j: (0, 0)),
    ],
    out_specs=pl.BlockSpec((4, BN_ROWS, Q), lambda j: (0, j, 0)),
    out_shape=jax.ShapeDtypeStruct((4, N, Q), jnp.float32),
)


def _edge_mlp_body(ea_ref, w_ref, b_ref, out_ref):
    ea = ea_ref[...]
    for q in range(4):
        y = _dot_bf16(ea, w_ref[q])
        out_ref[q] = jnp.maximum(y + b_ref[q], 0.0)


_edge_mlp = pl.pallas_call(
    _edge_mlp_body,
    grid=(NEB8,),
    in_specs=[
        pl.BlockSpec((EB8, 32), lambda j: (j, 0)),
        pl.BlockSpec((4, 32, 128), lambda j: (0, 0, 0)),
        pl.BlockSpec((4, 1, 128), lambda j: (0, 0, 0)),
    ],
    out_specs=pl.BlockSpec((4, EB8, 128), lambda j: (0, j, 0)),
    out_shape=jax.ShapeDtypeStruct((4, E8, 128), jnp.float32),
)


def _mlp_body(h4_ref, agg_ref, w1_ref, b1_ref, w2_ref, b2_ref, eps_ref,
              z_ref, stats_ref, acc_ref):
    j = pl.program_id(0)
    h = jnp.concatenate([h4_ref[0], h4_ref[1], h4_ref[2], h4_ref[3]], axis=1)
    agg = jnp.concatenate([agg_ref[0], agg_ref[1], agg_ref[2], agg_ref[3]],
                          axis=1)
    hc = (1.0 + eps_ref[0, 0]) * h + agg
    t = _dot_bf16(hc, w1_ref[...])
    t = jnp.maximum(t + b1_ref[...], 0.0)
    t = _dot_bf16(t, w2_ref[...])
    t = t + b2_ref[...]
    z_ref[...] = t

    @pl.when(j == 0)
    def _():
        acc_ref[...] = jnp.zeros_like(acc_ref)

    acc_ref[0:1, :] += jnp.sum(t, axis=0, keepdims=True)
    acc_ref[1:2, :] += jnp.sum(t * t, axis=0, keepdims=True)

    @pl.when(j == NB - 1)
    def _():
        stats_ref[...] = acc_ref[...]


_mlp = pl.pallas_call(
    _mlp_body,
    grid=(NB,),
    in_specs=[
        pl.BlockSpec((4, BN_ROWS, Q), lambda j: (0, j, 0)),
        pl.BlockSpec((4, BN_ROWS, Q), lambda j: (0, j, 0)),
        pl.BlockSpec((D, 2 * D), lambda j: (0, 0)),
        pl.BlockSpec((1, 2 * D), lambda j: (0, 0)),
        pl.BlockSpec((2 * D, D), lambda j: (0, 0)),
        pl.BlockSpec((1, D), lambda j: (0, 0)),
        pl.BlockSpec((1, 1), lambda j: (0, 0)),
    ],
    out_specs=[
        pl.BlockSpec((BN_ROWS, D), lambda j: (j, 0)),
        pl.BlockSpec((2, D), lambda j: (0, 0)),
    ],
    out_shape=[
        jax.ShapeDtypeStruct((N, D), jnp.float32),
        jax.ShapeDtypeStruct((2, D), jnp.float32),
    ],
    scratch_shapes=[pltpu.VMEM((2, D), jnp.float32)],
)


def _bn_body(z_ref, stats_ref, g_ref, be_ref, out_ref, *, relu, split):
    mean = stats_ref[0:1, :] / N
    var = stats_ref[1:2, :] / N - mean * mean
    y = ((z_ref[...] - mean) / jnp.sqrt(var + 1e-5)) * g_ref[...] + be_ref[...]
    if relu:
        y = jnp.maximum(y, 0.0)
    if split:
        _split4(y, out_ref)
    else:
        out_ref[...] = y


def _make_bn(relu, split):
    if split:
        out_specs = pl.BlockSpec((4, BN_ROWS, Q), lambda j: (0, j, 0))
        out_shape = jax.ShapeDtypeStruct((4, N, Q), jnp.float32)
    else:
        out_specs = pl.BlockSpec((BN_ROWS, D), lambda j: (j, 0))
        out_shape = jax.ShapeDtypeStruct((N, D), jnp.float32)
    return pl.pallas_call(
        functools.partial(_bn_body, relu=relu, split=split),
        grid=(NB,),
        in_specs=[
            pl.BlockSpec((BN_ROWS, D), lambda j: (j, 0)),
            pl.BlockSpec((2, D), lambda j: (0, 0)),
            pl.BlockSpec((1, D), lambda j: (0, 0)),
            pl.BlockSpec((1, D), lambda j: (0, 0)),
        ],
        out_specs=out_specs,
        out_shape=out_shape,
    )


_bn_mid = _make_bn(relu=True, split=True)
_bn_last = _make_bn(relu=False, split=False)


def kernel(x, edge_index, edge_attr, params):
    src = edge_index[0].astype(jnp.int32)
    dst = edge_index[1].astype(jnp.int32)
    bucket = dst // BW
    counts = jnp.bincount(bucket, length=32)
    nchunks = (counts + (CHUNK - 1)) // CHUNK
    off = jnp.concatenate([jnp.zeros((1,), jnp.int32),
                           jnp.cumsum(nchunks * CHUNK)[:-1].astype(jnp.int32)])
    onehot = (bucket[:, None] == jnp.arange(32)[None, :]).astype(jnp.int32)
    rank = jnp.take_along_axis(jnp.cumsum(onehot, axis=0) - onehot,
                               bucket[:, None], axis=1)[:, 0]
    pos = off[bucket] + rank
    src_p = jnp.zeros((E_CAP,), jnp.int32).at[pos].set(src)
    dst_p = jnp.full((E_CAP,), DUMP, jnp.int32).at[pos].set(dst)
    ea_b = jnp.zeros((E_CAP, 4), jnp.float32).at[pos].set(edge_attr)
    rot = (jnp.arange(32)[:, None] + jnp.arange(16)[None, :]) % 32
    meta = jnp.stack([nchunks.astype(jnp.int32)[rot],
                      off[rot]]).reshape(1024)
    ea8 = ea_b.reshape(E8, 32)
    eye8 = jnp.eye(8, dtype=jnp.float32)

    h4 = _embed(x, params['W_emb'], params['b_emb'].reshape(1, D))
    out = None
    for l, p in enumerate(params['layers']):
        w8 = jnp.stack([jnp.kron(eye8, p['W_e'][:, q * Q:(q + 1) * Q])
                        for q in range(4)])
        b8 = jnp.stack([jnp.tile(p['b_e'][q * Q:(q + 1) * Q], 8)[None, :]
                        for q in range(4)])
        e4 = _edge_mlp(ea8, w8, b8).reshape(4 * E_CAP, Q)
        agg = _sc_edge(h4.reshape(4 * N, Q), e4, src_p, dst_p, meta)
        z, stats = _mlp(h4, agg.reshape(4, NPAD, Q),
                        p['W1'], p['b1'].reshape(1, 2 * D),
                        p['W2'], p['b2'].reshape(1, D),
                        p['eps'].reshape(1, 1))
        gam = p['gamma'].reshape(1, D)
        bet = p['beta'].reshape(1, D)
        if l < NL - 1:
            h4 = _bn_mid(z, stats, gam, bet)
        else:
            out = _bn_last(z, stats, gam, bet)
    return out
